# baseline (speedup 1.0000x reference)
import functools

import jax
import jax.numpy as jnp
from jax.experimental import pallas as pl
from jax.experimental.pallas import tpu as pltpu

D_MODEL = 1024
D_FF = 4096
N_EXPERTS = 8
TOP_K = 2
BF = 512


def _moe_dense_kernel(wcol_ref, x_ref, wg_ref, wu_ref, wd_ref, out_ref):
    e = pl.program_id(0)
    j = pl.program_id(1)
    x = x_ref[...]
    g = jnp.dot(x, wg_ref[0], preferred_element_type=jnp.float32)
    u = jnp.dot(x, wu_ref[0], preferred_element_type=jnp.float32)
    w = wcol_ref[0]
    g = g * w
    u = u * w
    h = (g * jax.nn.sigmoid(g) * u).astype(jnp.bfloat16)
    y = jnp.dot(h, wd_ref[0], preferred_element_type=jnp.float32)

    @pl.when(jnp.logical_and(e == 0, j == 0))
    def _init():
        out_ref[...] = jnp.zeros_like(out_ref)

    out_ref[...] += y


def kernel(hidden_states, router_w, w_gate, w_up, w_down, sw_gate, sw_up, sw_down):
    b, s, d = hidden_states.shape
    t = b * s
    x = hidden_states.reshape(t, d)

    router_logits = x.astype(jnp.float32) @ router_w.astype(jnp.float32)
    affinities = jax.nn.sigmoid(router_logits)
    top_vals, top_idx = jax.lax.top_k(affinities, TOP_K)

    wdense = (
        jnp.zeros((t, N_EXPERTS), jnp.float32)
        .at[jnp.arange(t)[:, None], top_idx]
        .add(top_vals)
    )
    wcol = jnp.concatenate([jnp.ones((t, 1), jnp.float32), wdense], axis=1)
    wcol = wcol.T.reshape(N_EXPERTS + 1, t, 1)

    wg_all = jnp.concatenate([sw_gate[None], w_gate], 0).astype(jnp.bfloat16)
    wu_all = jnp.concatenate([sw_up[None], w_up], 0).astype(jnp.bfloat16)
    wd_all = jnp.concatenate([sw_down[None], w_down], 0).astype(jnp.bfloat16)
    x16 = x.astype(jnp.bfloat16)

    nj = D_FF // BF
    out = pl.pallas_call(
        _moe_dense_kernel,
        grid=(N_EXPERTS + 1, nj),
        in_specs=[
            pl.BlockSpec((1, t, 1), lambda e, j: (e, 0, 0)),
            pl.BlockSpec((t, d), lambda e, j: (0, 0)),
            pl.BlockSpec((1, d, BF), lambda e, j: (e, 0, j)),
            pl.BlockSpec((1, d, BF), lambda e, j: (e, 0, j)),
            pl.BlockSpec((1, BF, d), lambda e, j: (e, j, 0)),
        ],
        out_specs=pl.BlockSpec((t, d), lambda e, j: (0, 0)),
        out_shape=jax.ShapeDtypeStruct((t, d), jnp.float32),
        compiler_params=pltpu.CompilerParams(
            dimension_semantics=("arbitrary", "arbitrary"),
        ),
    )(wcol, x16, wg_all, wu_all, wd_all)

    return out.reshape(b, s, d)


# speedup vs baseline: 1.3853x; 1.3853x over previous
import functools

import jax
import jax.numpy as jnp
from jax import lax
from jax.experimental import pallas as pl
from jax.experimental.pallas import tpu as pltpu
from jax.experimental.pallas import tpu_sc as plsc

D_MODEL = 1024
D_FF = 4096
N_EXPERTS = 8
TOP_K = 2
BM = 256
BF = 2048
BFS = 512

_SC_CORES = 2
_SC_SUBCORES = 16
_SC_WORKERS = _SC_CORES * _SC_SUBCORES


def _make_sc_row_gather(n_rows, n_out, d):
    rpw = n_out // _SC_WORKERS
    chunk = rpw
    while chunk * d * 4 > 384 * 1024:
        chunk //= 2
    nch = rpw // chunk
    mesh = plsc.VectorSubcoreMesh(core_axis_name="c", subcore_axis_name="s")

    @functools.partial(
        pl.kernel,
        mesh=mesh,
        out_type=jax.ShapeDtypeStruct((n_out, d), jnp.float32),
        scratch_types=[
            pltpu.VMEM((chunk,), jnp.int32),
            pltpu.VMEM((chunk, d), jnp.float32),
            pltpu.SemaphoreType.DMA,
        ],
    )
    def gather_k(table_hbm, idx_hbm, out_hbm, idx_v, rows_v, sem):
        wid = lax.axis_index("s") * _SC_CORES + lax.axis_index("c")
        base0 = wid * rpw
        for c in range(nch):
            base = base0 + c * chunk
            pltpu.sync_copy(idx_hbm.at[pl.ds(base, chunk)], idx_v)
            pltpu.async_copy(table_hbm.at[idx_v], rows_v, sem).wait()
            pltpu.sync_copy(rows_v, out_hbm.at[pl.ds(base, chunk)])

    return gather_k


def _routed_kernel(emap_ref, vals_ref, xs_ref, wg_ref, wu_ref, wd_ref, out_ref):
    j = pl.program_id(1)
    xb = xs_ref[...].astype(jnp.bfloat16)
    g = jnp.dot(xb, wg_ref[0], preferred_element_type=jnp.float32)
    u = jnp.dot(xb, wu_ref[0], preferred_element_type=jnp.float32)
    v = vals_ref[0]
    g = g * v
    u = u * v
    h = (g * jax.nn.sigmoid(g) * u).astype(jnp.bfloat16)

    @pl.when(j == 0)
    def _():
        out_ref[...] = jnp.zeros_like(out_ref)

    out_ref[...] += jnp.dot(h, wd_ref[0], preferred_element_type=jnp.float32)


def _shared_combine_kernel(x_ref, wg_ref, wu_ref, wd_ref, ga_ref, gb_ref,
                           out_ref):
    j = pl.program_id(0)
    x = x_ref[...]
    g = jnp.dot(x, wg_ref[...], preferred_element_type=jnp.float32)
    u = jnp.dot(x, wu_ref[...], preferred_element_type=jnp.float32)
    h = (g * jax.nn.sigmoid(g) * u).astype(jnp.bfloat16)

    @pl.when(j == 0)
    def _():
        out_ref[...] = jnp.zeros_like(out_ref)

    out_ref[...] += jnp.dot(h, wd_ref[...], preferred_element_type=jnp.float32)
    nd = D_MODEL // (D_FF // BFS)
    out_ref[:, pl.ds(j * nd, nd)] += ga_ref[...] + gb_ref[...]


def kernel(hidden_states, router_w, w_gate, w_up, w_down, sw_gate, sw_up,
           sw_down):
    b, s, d = hidden_states.shape
    t = b * s
    x = hidden_states.reshape(t, d)

    router_logits = x.astype(jnp.float32) @ router_w.astype(jnp.float32)
    affinities = jax.nn.sigmoid(router_logits)
    top_vals, top_idx = jax.lax.top_k(affinities, TOP_K)

    n_pairs = t * TOP_K
    ep = top_idx.reshape(n_pairs)
    pv = top_vals.reshape(n_pairs)
    onehot = (ep[:, None] == jnp.arange(N_EXPERTS)[None, :]).astype(jnp.int32)
    csum = jnp.cumsum(onehot, axis=0)
    rank = jnp.take_along_axis(csum, ep[:, None], axis=1)[:, 0] - 1
    counts = csum[-1]
    padded = ((counts + BM - 1) // BM) * BM
    seg_start = jnp.concatenate([jnp.zeros((1,), jnp.int32),
                                 jnp.cumsum(padded)[:-1].astype(jnp.int32)])
    slot = (seg_start[ep] + rank).astype(jnp.int32)
    n_slots = n_pairs + N_EXPERTS * BM
    tok_of_slot = jnp.zeros((n_slots,), jnp.int32).at[slot].set(
        jnp.arange(n_pairs, dtype=jnp.int32) // TOP_K)
    val_of_slot = jnp.zeros((n_slots,), jnp.float32).at[slot].set(pv)
    nt = n_slots // BM
    tile_base = jnp.arange(nt, dtype=jnp.int32) * BM
    seg_end = seg_start + padded
    emap = jnp.sum((tile_base[:, None] >= seg_end[None, :]).astype(jnp.int32),
                   axis=1)
    emap = jnp.minimum(emap, N_EXPERTS - 1)
    poscat = slot.reshape(t, TOP_K).T.reshape(2 * t)

    xs = _make_sc_row_gather(t, n_slots, d)(x, tok_of_slot)

    vals3 = val_of_slot.reshape(nt, BM, 1)
    wg16 = w_gate.astype(jnp.bfloat16)
    wu16 = w_up.astype(jnp.bfloat16)
    wd16 = w_down.astype(jnp.bfloat16)
    nj = D_FF // BF
    y_routed = pl.pallas_call(
        _routed_kernel,
        grid_spec=pltpu.PrefetchScalarGridSpec(
            num_scalar_prefetch=1,
            grid=(nt, nj),
            in_specs=[
                pl.BlockSpec((1, BM, 1), lambda i, j, em: (i, 0, 0)),
                pl.BlockSpec((BM, d), lambda i, j, em: (i, 0)),
                pl.BlockSpec((1, d, BF), lambda i, j, em: (em[i], 0, j)),
                pl.BlockSpec((1, d, BF), lambda i, j, em: (em[i], 0, j)),
                pl.BlockSpec((1, BF, d), lambda i, j, em: (em[i], j, 0)),
            ],
            out_specs=pl.BlockSpec((BM, d), lambda i, j, em: (i, 0)),
        ),
        out_shape=jax.ShapeDtypeStruct((n_slots, d), jnp.float32),
        compiler_params=pltpu.CompilerParams(
            dimension_semantics=("arbitrary", "arbitrary"),
        ),
    )(emap, vals3, xs, wg16, wu16, wd16)

    gcat = _make_sc_row_gather(n_slots, 2 * t, d)(y_routed, poscat)

    njs = D_FF // BFS
    nd = d // njs
    out = pl.pallas_call(
        _shared_combine_kernel,
        grid=(njs,),
        in_specs=[
            pl.BlockSpec((t, d), lambda j: (0, 0)),
            pl.BlockSpec((d, BFS), lambda j: (0, j)),
            pl.BlockSpec((d, BFS), lambda j: (0, j)),
            pl.BlockSpec((BFS, d), lambda j: (j, 0)),
            pl.BlockSpec((t, nd), lambda j: (0, j)),
            pl.BlockSpec((t, nd), lambda j: (1, j)),
        ],
        out_specs=pl.BlockSpec((t, d), lambda j: (0, 0)),
        out_shape=jax.ShapeDtypeStruct((t, d), jnp.float32),
        compiler_params=pltpu.CompilerParams(
            dimension_semantics=("arbitrary",),
        ),
    )(x.astype(jnp.bfloat16), sw_gate.astype(jnp.bfloat16),
      sw_up.astype(jnp.bfloat16), sw_down.astype(jnp.bfloat16),
      gcat, gcat)

    return out.reshape(b, s, d)


# speedup vs baseline: 1.5826x; 1.1424x over previous
import functools

import jax
import jax.numpy as jnp
from jax import lax
from jax.experimental import pallas as pl
from jax.experimental.pallas import tpu as pltpu
from jax.experimental.pallas import tpu_sc as plsc

D_MODEL = 1024
D_FF = 4096
N_EXPERTS = 8
TOP_K = 2
BM = 256
BF = 2048
BFS = 512

_SC_CORES = 2
_SC_SUBCORES = 16
_SC_WORKERS = _SC_CORES * _SC_SUBCORES


def _make_sc_row_gather(n_rows, n_out, d):
    rpw = n_out // _SC_WORKERS
    chunk = rpw
    while chunk * d * 4 > 384 * 1024:
        chunk //= 2
    nch = rpw // chunk
    mesh = plsc.VectorSubcoreMesh(core_axis_name="c", subcore_axis_name="s")

    @functools.partial(
        pl.kernel,
        mesh=mesh,
        out_type=jax.ShapeDtypeStruct((n_out, d), jnp.float32),
        scratch_types=[
            pltpu.VMEM((chunk,), jnp.int32),
            pltpu.VMEM((chunk, d), jnp.float32),
            pltpu.SemaphoreType.DMA,
        ],
    )
    def gather_k(table_hbm, idx_hbm, out_hbm, idx_v, rows_v, sem):
        wid = lax.axis_index("s") * _SC_CORES + lax.axis_index("c")
        base0 = wid * rpw
        for c in range(nch):
            base = base0 + c * chunk
            pltpu.sync_copy(idx_hbm.at[pl.ds(base, chunk)], idx_v)
            pltpu.async_copy(table_hbm.at[idx_v], rows_v, sem).wait()
            pltpu.sync_copy(rows_v, out_hbm.at[pl.ds(base, chunk)])

    return gather_k


def _routed_kernel(emap_ref, vals_ref, tok_ref, x_ref, wg_ref, wu_ref, wd_ref,
                   out_ref, xg_ref):
    j = pl.program_id(1)

    @pl.when(j == 0)
    def _gather():
        tok = tok_ref[0]
        lane = jax.lax.broadcasted_iota(jnp.int32, (BM, x_ref.shape[0]), 1)
        oh = (lane == tok).astype(jnp.bfloat16)
        xg_ref[...] = jnp.dot(
            oh, x_ref[...], preferred_element_type=jnp.float32
        ).astype(jnp.bfloat16)

    xb = xg_ref[...]
    g = jnp.dot(xb, wg_ref[0], preferred_element_type=jnp.float32)
    u = jnp.dot(xb, wu_ref[0], preferred_element_type=jnp.float32)
    v = vals_ref[0]
    g = g * v
    u = u * v
    h = (g * jax.nn.sigmoid(g) * u).astype(jnp.bfloat16)

    @pl.when(j == 0)
    def _():
        out_ref[...] = jnp.zeros_like(out_ref)

    out_ref[...] += jnp.dot(h, wd_ref[0], preferred_element_type=jnp.float32)


def _shared_combine_kernel(x_ref, wg_ref, wu_ref, wd_ref, ga_ref, gb_ref,
                           out_ref):
    j = pl.program_id(0)
    x = x_ref[...]
    g = jnp.dot(x, wg_ref[...], preferred_element_type=jnp.float32)
    u = jnp.dot(x, wu_ref[...], preferred_element_type=jnp.float32)
    h = (g * jax.nn.sigmoid(g) * u).astype(jnp.bfloat16)

    @pl.when(j == 0)
    def _():
        out_ref[...] = jnp.zeros_like(out_ref)

    out_ref[...] += jnp.dot(h, wd_ref[...], preferred_element_type=jnp.float32)
    nd = D_MODEL // (D_FF // BFS)
    out_ref[:, pl.ds(j * nd, nd)] += ga_ref[...] + gb_ref[...]


def kernel(hidden_states, router_w, w_gate, w_up, w_down, sw_gate, sw_up,
           sw_down):
    b, s, d = hidden_states.shape
    t = b * s
    x = hidden_states.reshape(t, d)

    router_logits = x.astype(jnp.float32) @ router_w.astype(jnp.float32)
    affinities = jax.nn.sigmoid(router_logits)
    top_vals, top_idx = jax.lax.top_k(affinities, TOP_K)

    n_pairs = t * TOP_K
    ep = top_idx.reshape(n_pairs)
    pv = top_vals.reshape(n_pairs)
    onehot = (ep[:, None] == jnp.arange(N_EXPERTS)[None, :]).astype(jnp.int32)
    csum = jnp.cumsum(onehot, axis=0)
    rank = jnp.take_along_axis(csum, ep[:, None], axis=1)[:, 0] - 1
    counts = csum[-1]
    padded = ((counts + BM - 1) // BM) * BM
    seg_start = jnp.concatenate([jnp.zeros((1,), jnp.int32),
                                 jnp.cumsum(padded)[:-1].astype(jnp.int32)])
    slot = (seg_start[ep] + rank).astype(jnp.int32)
    n_slots = n_pairs + N_EXPERTS * BM
    tok_of_slot = jnp.zeros((n_slots,), jnp.int32).at[slot].set(
        jnp.arange(n_pairs, dtype=jnp.int32) // TOP_K)
    val_of_slot = jnp.zeros((n_slots,), jnp.float32).at[slot].set(pv)
    nt = n_slots // BM
    tile_base = jnp.arange(nt, dtype=jnp.int32) * BM
    seg_end = seg_start + padded
    emap = jnp.sum((tile_base[:, None] >= seg_end[None, :]).astype(jnp.int32),
                   axis=1)
    emap = jnp.minimum(emap, N_EXPERTS - 1)
    poscat = slot.reshape(t, TOP_K).T.reshape(2 * t)

    toks3 = tok_of_slot.reshape(nt, BM, 1)
    vals3 = val_of_slot.reshape(nt, BM, 1)
    wg16 = w_gate.astype(jnp.bfloat16)
    wu16 = w_up.astype(jnp.bfloat16)
    wd16 = w_down.astype(jnp.bfloat16)
    nj = D_FF // BF
    y_routed = pl.pallas_call(
        _routed_kernel,
        grid_spec=pltpu.PrefetchScalarGridSpec(
            num_scalar_prefetch=1,
            grid=(nt, nj),
            in_specs=[
                pl.BlockSpec((1, BM, 1), lambda i, j, em: (i, 0, 0)),
                pl.BlockSpec((1, BM, 1), lambda i, j, em: (i, 0, 0)),
                pl.BlockSpec((t, d), lambda i, j, em: (0, 0)),
                pl.BlockSpec((1, d, BF), lambda i, j, em: (em[i], 0, j)),
                pl.BlockSpec((1, d, BF), lambda i, j, em: (em[i], 0, j)),
                pl.BlockSpec((1, BF, d), lambda i, j, em: (em[i], j, 0)),
            ],
            out_specs=pl.BlockSpec((BM, d), lambda i, j, em: (i, 0)),
            scratch_shapes=[pltpu.VMEM((BM, d), jnp.bfloat16)],
        ),
        out_shape=jax.ShapeDtypeStruct((n_slots, d), jnp.float32),
        compiler_params=pltpu.CompilerParams(
            dimension_semantics=("arbitrary", "arbitrary"),
        ),
    )(emap, vals3, toks3, x.astype(jnp.bfloat16), wg16, wu16, wd16)

    gcat = _make_sc_row_gather(n_slots, 2 * t, d)(y_routed, poscat)

    njs = D_FF // BFS
    nd = d // njs
    out = pl.pallas_call(
        _shared_combine_kernel,
        grid=(njs,),
        in_specs=[
            pl.BlockSpec((t, d), lambda j: (0, 0)),
            pl.BlockSpec((d, BFS), lambda j: (0, j)),
            pl.BlockSpec((d, BFS), lambda j: (0, j)),
            pl.BlockSpec((BFS, d), lambda j: (j, 0)),
            pl.BlockSpec((t, nd), lambda j: (0, j)),
            pl.BlockSpec((t, nd), lambda j: (1, j)),
        ],
        out_specs=pl.BlockSpec((t, d), lambda j: (0, 0)),
        out_shape=jax.ShapeDtypeStruct((t, d), jnp.float32),
        compiler_params=pltpu.CompilerParams(
            dimension_semantics=("arbitrary",),
        ),
    )(x.astype(jnp.bfloat16), sw_gate.astype(jnp.bfloat16),
      sw_up.astype(jnp.bfloat16), sw_down.astype(jnp.bfloat16),
      gcat, gcat)

    return out.reshape(b, s, d)


# speedup vs baseline: 1.5908x; 1.0052x over previous
import functools

import jax
import jax.numpy as jnp
from jax import lax
from jax.experimental import pallas as pl
from jax.experimental.pallas import tpu as pltpu
from jax.experimental.pallas import tpu_sc as plsc

D_MODEL = 1024
D_FF = 4096
N_EXPERTS = 8
TOP_K = 2
BM = 256
BF = 2048
BFS = 512

_SC_CORES = 2
_SC_SUBCORES = 16
_SC_WORKERS = _SC_CORES * _SC_SUBCORES


def _make_sc_row_gather(n_rows, n_out, d):
    rpw = n_out // _SC_WORKERS
    chunk = rpw
    while chunk * d * 4 > 384 * 1024:
        chunk //= 2
    nch = rpw // chunk
    mesh = plsc.VectorSubcoreMesh(core_axis_name="c", subcore_axis_name="s")

    @functools.partial(
        pl.kernel,
        mesh=mesh,
        out_type=jax.ShapeDtypeStruct((n_out, d), jnp.float32),
        scratch_types=[
            pltpu.VMEM((chunk,), jnp.int32),
            pltpu.VMEM((chunk, d), jnp.float32),
            pltpu.SemaphoreType.DMA,
        ],
    )
    def gather_k(table_hbm, idx_hbm, out_hbm, idx_v, rows_v, sem):
        wid = lax.axis_index("s") * _SC_CORES + lax.axis_index("c")
        base0 = wid * rpw
        for c in range(nch):
            base = base0 + c * chunk
            pltpu.sync_copy(idx_hbm.at[pl.ds(base, chunk)], idx_v)
            pltpu.async_copy(table_hbm.at[idx_v], rows_v, sem).wait()
            pltpu.sync_copy(rows_v, out_hbm.at[pl.ds(base, chunk)])

    return gather_k


def _routed_kernel(emap_ref, vals_ref, tok_ref, x_ref, wg_ref, wu_ref, wd_ref,
                   out_ref, xg_ref):
    j = pl.program_id(1)

    @pl.when(j == 0)
    def _gather():
        tok = tok_ref[0]
        lane = jax.lax.broadcasted_iota(jnp.int32, (BM, x_ref.shape[0]), 1)
        oh = (lane == tok).astype(jnp.bfloat16)
        xg_ref[...] = jnp.dot(
            oh, x_ref[...], preferred_element_type=jnp.float32
        ).astype(jnp.bfloat16)

    xb = xg_ref[...]
    g = jnp.dot(xb, wg_ref[0], preferred_element_type=jnp.float32)
    u = jnp.dot(xb, wu_ref[0], preferred_element_type=jnp.float32)
    v = vals_ref[0]
    g = g * v
    u = u * v
    h = (g * jax.nn.sigmoid(g) * u).astype(jnp.bfloat16)

    @pl.when(j == 0)
    def _():
        out_ref[...] = jnp.zeros_like(out_ref)

    out_ref[...] += jnp.dot(h, wd_ref[0], preferred_element_type=jnp.float32)


def _shared_kernel(x_ref, wg_ref, wu_ref, wd_ref, out_ref):
    j = pl.program_id(0)
    x = x_ref[...]
    g = jnp.dot(x, wg_ref[...], preferred_element_type=jnp.float32)
    u = jnp.dot(x, wu_ref[...], preferred_element_type=jnp.float32)
    h = (g * jax.nn.sigmoid(g) * u).astype(jnp.bfloat16)

    @pl.when(j == 0)
    def _():
        out_ref[...] = jnp.zeros_like(out_ref)

    out_ref[...] += jnp.dot(h, wd_ref[...], preferred_element_type=jnp.float32)


def _add3_kernel(a_ref, b_ref, c_ref, out_ref):
    out_ref[...] = a_ref[...] + b_ref[...] + c_ref[...]


def kernel(hidden_states, router_w, w_gate, w_up, w_down, sw_gate, sw_up,
           sw_down):
    b, s, d = hidden_states.shape
    t = b * s
    x = hidden_states.reshape(t, d)

    router_logits = x.astype(jnp.float32) @ router_w.astype(jnp.float32)
    affinities = jax.nn.sigmoid(router_logits)
    top_vals, top_idx = jax.lax.top_k(affinities, TOP_K)

    n_pairs = t * TOP_K
    ep = top_idx.reshape(n_pairs)
    pv = top_vals.reshape(n_pairs)
    onehot = (ep[:, None] == jnp.arange(N_EXPERTS)[None, :]).astype(jnp.float32)
    ch = 128
    ncha = n_pairs // ch
    oh3 = onehot.reshape(ncha, ch, N_EXPERTS)
    r_i = jnp.arange(ch)
    tri_inc = (r_i[:, None] >= r_i[None, :]).astype(jnp.float32)
    within = jnp.einsum('rc,kce->kre', tri_inc, oh3,
                        preferred_element_type=jnp.float32)
    chunk_tot = within[:, -1, :]
    r_k = jnp.arange(ncha)
    tri_exc = (r_k[:, None] > r_k[None, :]).astype(jnp.float32)
    chunk_pre = jnp.einsum('kc,ce->ke', tri_exc, chunk_tot,
                           preferred_element_type=jnp.float32)
    csum3 = within + chunk_pre[:, None, :]
    rank = jnp.sum(oh3 * csum3, axis=-1).reshape(n_pairs).astype(jnp.int32) - 1
    counts = jnp.sum(onehot, axis=0).astype(jnp.int32)
    padded = ((counts + BM - 1) // BM) * BM
    seg_start = jnp.concatenate([jnp.zeros((1,), jnp.int32),
                                 jnp.cumsum(padded)[:-1].astype(jnp.int32)])
    slot = (seg_start[ep] + rank).astype(jnp.int32)
    n_slots = n_pairs + N_EXPERTS * BM
    tok_of_slot = jnp.zeros((n_slots,), jnp.int32).at[slot].set(
        jnp.arange(n_pairs, dtype=jnp.int32) // TOP_K)
    val_of_slot = jnp.zeros((n_slots,), jnp.float32).at[slot].set(pv)
    nt = n_slots // BM
    tile_base = jnp.arange(nt, dtype=jnp.int32) * BM
    seg_end = seg_start + padded
    emap = jnp.sum((tile_base[:, None] >= seg_end[None, :]).astype(jnp.int32),
                   axis=1)
    emap = jnp.minimum(emap, N_EXPERTS - 1)
    poscat = slot.reshape(t, TOP_K).T.reshape(2 * t)

    toks3 = tok_of_slot.reshape(nt, BM, 1)
    vals3 = val_of_slot.reshape(nt, BM, 1)
    wg16 = w_gate.astype(jnp.bfloat16)
    wu16 = w_up.astype(jnp.bfloat16)
    wd16 = w_down.astype(jnp.bfloat16)
    nj = D_FF // BF
    y_routed = pl.pallas_call(
        _routed_kernel,
        grid_spec=pltpu.PrefetchScalarGridSpec(
            num_scalar_prefetch=1,
            grid=(nt, nj),
            in_specs=[
                pl.BlockSpec((1, BM, 1), lambda i, j, em: (i, 0, 0)),
                pl.BlockSpec((1, BM, 1), lambda i, j, em: (i, 0, 0)),
                pl.BlockSpec((t, d), lambda i, j, em: (0, 0)),
                pl.BlockSpec((1, d, BF), lambda i, j, em: (em[i], 0, j)),
                pl.BlockSpec((1, d, BF), lambda i, j, em: (em[i], 0, j)),
                pl.BlockSpec((1, BF, d), lambda i, j, em: (em[i], j, 0)),
            ],
            out_specs=pl.BlockSpec((BM, d), lambda i, j, em: (i, 0)),
            scratch_shapes=[pltpu.VMEM((BM, d), jnp.bfloat16)],
        ),
        out_shape=jax.ShapeDtypeStruct((n_slots, d), jnp.float32),
        compiler_params=pltpu.CompilerParams(
            dimension_semantics=("arbitrary", "arbitrary"),
        ),
    )(emap, vals3, toks3, x.astype(jnp.bfloat16), wg16, wu16, wd16)

    gcat = _make_sc_row_gather(n_slots, 2 * t, d)(y_routed, poscat)

    njs = D_FF // BFS
    ys = pl.pallas_call(
        _shared_kernel,
        grid=(njs,),
        in_specs=[
            pl.BlockSpec((t, d), lambda j: (0, 0)),
            pl.BlockSpec((d, BFS), lambda j: (0, j)),
            pl.BlockSpec((d, BFS), lambda j: (0, j)),
            pl.BlockSpec((BFS, d), lambda j: (j, 0)),
        ],
        out_specs=pl.BlockSpec((t, d), lambda j: (0, 0)),
        out_shape=jax.ShapeDtypeStruct((t, d), jnp.float32),
        compiler_params=pltpu.CompilerParams(
            dimension_semantics=("arbitrary",),
        ),
    )(x.astype(jnp.bfloat16), sw_gate.astype(jnp.bfloat16),
      sw_up.astype(jnp.bfloat16), sw_down.astype(jnp.bfloat16))

    brow = 512
    out = pl.pallas_call(
        _add3_kernel,
        grid=(t // brow,),
        in_specs=[
            pl.BlockSpec((brow, d), lambda i: (i, 0)),
            pl.BlockSpec((brow, d), lambda i: (i, 0)),
            pl.BlockSpec((brow, d), lambda i: (i + t // brow, 0)),
        ],
        out_specs=pl.BlockSpec((brow, d), lambda i: (i, 0)),
        out_shape=jax.ShapeDtypeStruct((t, d), jnp.float32),
    )(ys, gcat, gcat)

    return out.reshape(b, s, d)


# speedup vs baseline: 1.6674x; 1.0482x over previous
import functools

import jax
import jax.numpy as jnp
from jax import lax
from jax.experimental import pallas as pl
from jax.experimental.pallas import tpu as pltpu
from jax.experimental.pallas import tpu_sc as plsc

D_MODEL = 1024
D_FF = 4096
N_EXPERTS = 8
TOP_K = 2
BM = 256
BF = 2048
BFS = 512

_SC_CORES = 2
_SC_SUBCORES = 16
_SC_WORKERS = _SC_CORES * _SC_SUBCORES


def _make_sc_row_gather(n_rows, n_out, d):
    rpw = n_out // _SC_WORKERS
    chunk = rpw
    while chunk * d * 4 > 384 * 1024:
        chunk //= 2
    nch = rpw // chunk
    mesh = plsc.VectorSubcoreMesh(core_axis_name="c", subcore_axis_name="s")

    @functools.partial(
        pl.kernel,
        mesh=mesh,
        out_type=jax.ShapeDtypeStruct((n_out, d), jnp.float32),
        scratch_types=[
            pltpu.VMEM((chunk,), jnp.int32),
            pltpu.VMEM((chunk, d), jnp.float32),
            pltpu.SemaphoreType.DMA,
        ],
    )
    def gather_k(table_hbm, idx_hbm, out_hbm, idx_v, rows_v, sem):
        wid = lax.axis_index("s") * _SC_CORES + lax.axis_index("c")
        base0 = wid * rpw
        for c in range(nch):
            base = base0 + c * chunk
            pltpu.sync_copy(idx_hbm.at[pl.ds(base, chunk)], idx_v)
            pltpu.async_copy(table_hbm.at[idx_v], rows_v, sem).wait()
            pltpu.sync_copy(rows_v, out_hbm.at[pl.ds(base, chunk)])

    return gather_k


def _routed_kernel(emap_ref, meta_ref, x_ref, wg_ref, wu_ref, wd_ref,
                   out_ref, xg_ref):
    i = pl.program_id(0)
    j = pl.program_id(1)

    @pl.when(j == 0)
    def _gather():
        rowf = (jax.lax.broadcasted_iota(jnp.int32, (BM, 1), 0)
                + i * BM).astype(jnp.float32)
        m = meta_ref[...]
        oh = (jnp.where(m[0:1, :] == rowf, m[2:3, :], 0.0)
              + jnp.where(m[1:2, :] == rowf, m[3:4, :], 0.0)
              ).astype(jnp.bfloat16)
        xg_ref[...] = jnp.dot(
            oh, x_ref[...], preferred_element_type=jnp.float32
        ).astype(jnp.bfloat16)

    xb = xg_ref[...]
    g = jnp.dot(xb, wg_ref[0], preferred_element_type=jnp.float32)
    u = jnp.dot(xb, wu_ref[0], preferred_element_type=jnp.float32)
    h = (g * jax.nn.sigmoid(g) * u).astype(jnp.bfloat16)

    @pl.when(j == 0)
    def _():
        out_ref[...] = jnp.zeros_like(out_ref)

    out_ref[...] += jnp.dot(h, wd_ref[0], preferred_element_type=jnp.float32)


def _shared_kernel(x_ref, wg_ref, wu_ref, wd_ref, out_ref):
    j = pl.program_id(0)
    x = x_ref[...]
    g = jnp.dot(x, wg_ref[...], preferred_element_type=jnp.float32)
    u = jnp.dot(x, wu_ref[...], preferred_element_type=jnp.float32)
    h = (g * jax.nn.sigmoid(g) * u).astype(jnp.bfloat16)

    @pl.when(j == 0)
    def _():
        out_ref[...] = jnp.zeros_like(out_ref)

    out_ref[...] += jnp.dot(h, wd_ref[...], preferred_element_type=jnp.float32)


def _add3_kernel(a_ref, b_ref, out_ref):
    d = a_ref.shape[1]
    out_ref[...] = a_ref[...] + b_ref[:, :d] + b_ref[:, d:]


def kernel(hidden_states, router_w, w_gate, w_up, w_down, sw_gate, sw_up,
           sw_down):
    b, s, d = hidden_states.shape
    t = b * s
    x = hidden_states.reshape(t, d)

    router_logits = x.astype(jnp.float32) @ router_w.astype(jnp.float32)
    affinities = jax.nn.sigmoid(router_logits)
    top_vals, top_idx = jax.lax.top_k(affinities, TOP_K)

    n_pairs = t * TOP_K
    ep = top_idx.reshape(n_pairs)
    pv = top_vals.reshape(n_pairs)
    onehot = (ep[:, None] == jnp.arange(N_EXPERTS)[None, :]).astype(jnp.float32)
    ch = 128
    ncha = n_pairs // ch
    oh3 = onehot.reshape(ncha, ch, N_EXPERTS)
    r_i = jnp.arange(ch)
    tri_inc = (r_i[:, None] >= r_i[None, :]).astype(jnp.float32)
    within = jnp.einsum('rc,kce->kre', tri_inc, oh3,
                        preferred_element_type=jnp.float32)
    chunk_tot = within[:, -1, :]
    r_k = jnp.arange(ncha)
    tri_exc = (r_k[:, None] > r_k[None, :]).astype(jnp.float32)
    chunk_pre = jnp.einsum('kc,ce->ke', tri_exc, chunk_tot,
                           preferred_element_type=jnp.float32)
    csum3 = within + chunk_pre[:, None, :]
    rank = jnp.sum(oh3 * csum3, axis=-1).reshape(n_pairs) - 1.0
    counts = jnp.sum(onehot, axis=0).astype(jnp.int32)
    padded = ((counts + BM - 1) // BM) * BM
    seg_start = jnp.concatenate([jnp.zeros((1,), jnp.int32),
                                 jnp.cumsum(padded)[:-1].astype(jnp.int32)])
    slot_f = (jnp.sum(onehot * seg_start[None, :].astype(jnp.float32), axis=1)
              + rank)
    slot = slot_f.astype(jnp.int32)
    n_slots = n_pairs + N_EXPERTS * BM
    nt = n_slots // BM
    tile_base = jnp.arange(nt, dtype=jnp.int32) * BM
    seg_end = seg_start + padded
    emap = jnp.sum((tile_base[:, None] >= seg_end[None, :]).astype(jnp.int32),
                   axis=1)
    emap = jnp.minimum(emap, N_EXPERTS - 1)
    meta2 = jnp.concatenate(
        [slot_f.reshape(t, TOP_K).T, top_vals.T], axis=0)

    wg16 = w_gate.astype(jnp.bfloat16)
    wu16 = w_up.astype(jnp.bfloat16)
    wd16 = w_down.astype(jnp.bfloat16)
    nj = D_FF // BF
    y_routed = pl.pallas_call(
        _routed_kernel,
        grid_spec=pltpu.PrefetchScalarGridSpec(
            num_scalar_prefetch=1,
            grid=(nt, nj),
            in_specs=[
                pl.BlockSpec((4, t), lambda i, j, em: (0, 0)),
                pl.BlockSpec((t, d), lambda i, j, em: (0, 0)),
                pl.BlockSpec((1, d, BF), lambda i, j, em: (em[i], 0, j)),
                pl.BlockSpec((1, d, BF), lambda i, j, em: (em[i], 0, j)),
                pl.BlockSpec((1, BF, d), lambda i, j, em: (em[i], j, 0)),
            ],
            out_specs=pl.BlockSpec((BM, d), lambda i, j, em: (i, 0)),
            scratch_shapes=[pltpu.VMEM((BM, d), jnp.bfloat16)],
        ),
        out_shape=jax.ShapeDtypeStruct((n_slots, d), jnp.float32),
        compiler_params=pltpu.CompilerParams(
            dimension_semantics=("arbitrary", "arbitrary"),
        ),
    )(emap, meta2, x.astype(jnp.bfloat16), wg16, wu16, wd16)

    gcat = _make_sc_row_gather(n_slots, 2 * t, d)(y_routed, slot)

    njs = D_FF // BFS
    ys = pl.pallas_call(
        _shared_kernel,
        grid=(njs,),
        in_specs=[
            pl.BlockSpec((t, d), lambda j: (0, 0)),
            pl.BlockSpec((d, BFS), lambda j: (0, j)),
            pl.BlockSpec((d, BFS), lambda j: (0, j)),
            pl.BlockSpec((BFS, d), lambda j: (j, 0)),
        ],
        out_specs=pl.BlockSpec((t, d), lambda j: (0, 0)),
        out_shape=jax.ShapeDtypeStruct((t, d), jnp.float32),
        compiler_params=pltpu.CompilerParams(
            dimension_semantics=("arbitrary",),
        ),
    )(x.astype(jnp.bfloat16), sw_gate.astype(jnp.bfloat16),
      sw_up.astype(jnp.bfloat16), sw_down.astype(jnp.bfloat16))

    brow = 512
    gcat2 = gcat.reshape(t, TOP_K * d)
    out = pl.pallas_call(
        _add3_kernel,
        grid=(t // brow,),
        in_specs=[
            pl.BlockSpec((brow, d), lambda i: (i, 0)),
            pl.BlockSpec((brow, TOP_K * d), lambda i: (i, 0)),
        ],
        out_specs=pl.BlockSpec((brow, d), lambda i: (i, 0)),
        out_shape=jax.ShapeDtypeStruct((t, d), jnp.float32),
    )(ys, gcat2)

    return out.reshape(b, s, d)
